# baseline (speedup 1.0000x reference)
import functools

import jax
import jax.numpy as jnp
from jax import lax
from jax.experimental import pallas as pl
from jax.experimental.pallas import tpu as pltpu
from jax.experimental.pallas import tpu_sc as plsc

N_NODES = 10000
N_EDGES = 320000
D_IN = 128
D_HID = 256
D_OUT = 64
DA = 64

NC = 2
NS = 16
NT = NC * NS
CH = 128
NCH = -(-N_EDGES // (NT * CH))
E_PAD = NT * NCH * CH
RPT = 640
ACC_ROWS = NS * RPT
TST = 624


def _make_sc_agg(with_cnt):
  mesh = plsc.VectorSubcoreMesh(core_axis_name="c", subcore_axis_name="s")
  out_type = [jax.ShapeDtypeStruct((NC, ACC_ROWS, DA), jnp.float32)]
  scratch = [
      pltpu.VMEM((NCH, CH), jnp.int32),
      pltpu.VMEM((NCH, CH), jnp.int32),
      pltpu.VMEM((CH, DA), jnp.float32),
      pltpu.VMEM_SHARED((N_NODES, DA), jnp.float32),
      pltpu.VMEM_SHARED((ACC_ROWS, DA), jnp.float32),
      pltpu.SemaphoreType.DMA,
      pltpu.SemaphoreType.DMA,
  ]
  if with_cnt:
    out_type.append(jax.ShapeDtypeStruct((NC, ACC_ROWS), jnp.float32))
    scratch += [
        pltpu.VMEM((CH,), jnp.float32),
        pltpu.VMEM((RPT,), jnp.float32),
        pltpu.VMEM_SHARED((ACC_ROWS,), jnp.float32),
    ]

  n16 = DA // 16

  def body(table, idx_h, *rest):
    zeros16 = jnp.zeros((16,), jnp.float32)
    ones16 = jnp.ones((16,), jnp.float32)
    if with_cnt:
      (out_a, out_c, src_v, dst_v, rows, tab, acc, sem0, sem1,
       ones_v, zc_v, cacc) = rest
    else:
      (out_a, src_v, dst_v, rows, tab, acc, sem0, sem1) = rest
    cid = lax.axis_index("c")
    sid = lax.axis_index("s")
    tid = cid * NS + sid
    base = sid * RPT

    pltpu.async_copy(idx_h.at[tid, 0], src_v, sem0)
    pltpu.async_copy(idx_h.at[tid, 1], dst_v, sem1)
    pltpu.sync_copy(table.at[pl.ds(sid * TST, TST)],
                    tab.at[pl.ds(sid * TST, TST)])
    @pl.when(sid == 0)
    def _():
      pltpu.sync_copy(table.at[pl.ds(NS * TST, N_NODES - NS * TST)],
                      tab.at[pl.ds(NS * TST, N_NODES - NS * TST)])

    def zrow(r, _):
      for c in range(n16):
        rows[r, pl.ds(c * 16, 16)] = zeros16
      return 0
    lax.fori_loop(0, CH, zrow, 0)
    for k in range(RPT // CH):
      pltpu.sync_copy(rows, acc.at[pl.ds(base + k * CH, CH)])
    if with_cnt:
      def zc(i, _):
        ones_v[pl.ds(i * 16, 16)] = ones16
        return 0
      lax.fori_loop(0, CH // 16, zc, 0)
      def zc2(i, _):
        zc_v[pl.ds(i * 16, 16)] = zeros16
        return 0
      lax.fori_loop(0, RPT // 16, zc2, 0)
      pltpu.sync_copy(zc_v, cacc.at[pl.ds(base, RPT)])
    pltpu.make_async_copy(idx_h.at[tid, 0], src_v, sem0).wait()
    pltpu.make_async_copy(idx_h.at[tid, 1], dst_v, sem1).wait()
    plsc.subcore_barrier()

    def step(j, _):
      pltpu.async_copy(tab.at[src_v.at[j]], rows, sem0).wait()
      if with_cnt:
        pltpu.sync_copy(ones_v, cacc.at[dst_v.at[j]], add=True)
      pltpu.sync_copy(rows, acc.at[dst_v.at[j]], add=True)
      return 0
    lax.fori_loop(0, NCH, step, 0)
    plsc.subcore_barrier()

    pltpu.sync_copy(acc.at[pl.ds(base, RPT)], out_a.at[cid, pl.ds(base, RPT)])
    if with_cnt:
      pltpu.sync_copy(cacc.at[pl.ds(base, RPT)],
                      out_c.at[cid, pl.ds(base, RPT)])

  return pl.kernel(body, out_type=tuple(out_type), mesh=mesh,
                   scratch_types=tuple(scratch),
                   compiler_params=pltpu.CompilerParams(
                       use_tc_tiling_on_sc=False))


_sc_agg_cnt = _make_sc_agg(True)
_sc_agg = _make_sc_agg(False)

BR = 1000


def _dense_body(x, al0, al1, ah0, ah1, c0, c1, w1l, b1l, w1r, w2l, w2r,
                h_ref, p_ref, q_ref, ic_ref):
  c = jnp.maximum(c0[...] + c1[...], 1.0)
  mean = jnp.concatenate([al0[...] + al1[...], ah0[...] + ah1[...]],
                         axis=1) / c
  h = lax.dot_general(mean, w1l[...], (((1,), (0,)), ((), ())),
                      preferred_element_type=jnp.float32)
  h += lax.dot_general(x[...], w1r[...], (((1,), (0,)), ((), ())),
                       preferred_element_type=jnp.float32)
  h = jnp.maximum(h + b1l[...], 0.0)
  h_ref[...] = h
  p_ref[...] = lax.dot_general(h, w2l[...], (((1,), (0,)), ((), ())),
                               preferred_element_type=jnp.float32)
  q_ref[...] = lax.dot_general(h, w2r[...], (((1,), (0,)), ((), ())),
                               preferred_element_type=jnp.float32)
  ic_ref[...] = 1.0 / c


def _final_body(g0, g1, ic, q, b2l, out_ref):
  g = g0[...] + g1[...]
  out_ref[...] = g * ic[...] + q[...] + b2l[...]


def _row_blk(d):
  return pl.BlockSpec((BR, d), lambda i: (i, 0))


def _full_blk(r, d):
  return pl.BlockSpec((r, d), lambda i: (0, 0))


_dense_call = pl.pallas_call(
    _dense_body,
    grid=(N_NODES // BR,),
    in_specs=[
        _row_blk(D_IN),
        _row_blk(DA),
        _row_blk(DA),
        _row_blk(DA),
        _row_blk(DA),
        _row_blk(1),
        _row_blk(1),
        _full_blk(D_IN, D_HID),
        _full_blk(1, D_HID),
        _full_blk(D_IN, D_HID),
        _full_blk(D_HID, D_OUT),
        _full_blk(D_HID, D_OUT),
    ],
    out_specs=[
        _row_blk(D_HID),
        _row_blk(D_OUT),
        _row_blk(D_OUT),
        _row_blk(1),
    ],
    out_shape=[
        jax.ShapeDtypeStruct((N_NODES, D_HID), jnp.float32),
        jax.ShapeDtypeStruct((N_NODES, D_OUT), jnp.float32),
        jax.ShapeDtypeStruct((N_NODES, D_OUT), jnp.float32),
        jax.ShapeDtypeStruct((N_NODES, 1), jnp.float32),
    ],
)

_final_call = pl.pallas_call(
    _final_body,
    grid=(N_NODES // BR,),
    in_specs=[
        _row_blk(DA),
        _row_blk(DA),
        _row_blk(1),
        _row_blk(D_OUT),
        _full_blk(1, D_OUT),
    ],
    out_specs=_row_blk(D_OUT),
    out_shape=jax.ShapeDtypeStruct((N_NODES, D_OUT), jnp.float32),
)


@jax.jit
def _run(x, edge_index, W1l, b1l, W1r, W2l, b2l, W2r):
  src = edge_index[0].astype(jnp.int32)
  dst = edge_index[1].astype(jnp.int32)
  pad = E_PAD - N_EDGES
  src = jnp.concatenate([src, jnp.zeros((pad,), jnp.int32)])
  dst = jnp.concatenate([dst, jnp.full((pad,), N_NODES, jnp.int32)])
  src_r = src.reshape(NT, 1, NCH, CH)
  dst_r = dst.reshape(NT, 1, NCH, CH)
  idx = jnp.concatenate([src_r, dst_r], axis=1)

  al, cnt = _sc_agg_cnt(x[:, :DA], idx)
  (ah,) = _sc_agg(x[:, DA:], idx)
  c2 = cnt[:, :N_NODES, None]
  h, p, q, ic = _dense_call(x, al[0, :N_NODES], al[1, :N_NODES],
                            ah[0, :N_NODES], ah[1, :N_NODES],
                            c2[0], c2[1], W1l.T, b1l[None, :], W1r.T,
                            W2l.T, W2r.T)
  (g,) = _sc_agg(p, idx)
  return _final_call(g[0, :N_NODES], g[1, :N_NODES], ic, q, b2l[None, :])


def kernel(x, edge_index, W1l, b1l, W1r, W2l, b2l, W2r):
  return _run(x, edge_index, W1l, b1l, W1r, W2l, b2l, W2r)


# speedup vs baseline: 1.1074x; 1.1074x over previous
import functools

import jax
import jax.numpy as jnp
from jax import lax
from jax.experimental import pallas as pl
from jax.experimental.pallas import tpu as pltpu
from jax.experimental.pallas import tpu_sc as plsc

N_NODES = 10000
N_EDGES = 320000
D_IN = 128
D_HID = 256
D_OUT = 64
DA = 64

NC = 2
NS = 16
NT = NC * NS
CH = 128
NCH = -(-N_EDGES // (NT * CH))
E_PAD = NT * NCH * CH
RPT = 640
ACC_ROWS = NS * RPT
TST = 624


def _make_sc_agg(with_cnt, col_off=0, in_cols=DA):
  mesh = plsc.VectorSubcoreMesh(core_axis_name="c", subcore_axis_name="s")
  out_type = [jax.ShapeDtypeStruct((NC, ACC_ROWS, DA), jnp.float32)]
  scratch = [
      pltpu.VMEM((NCH, CH), jnp.int32),
      pltpu.VMEM((NCH, CH), jnp.int32),
      pltpu.VMEM((CH, DA), jnp.float32),
      pltpu.VMEM_SHARED((N_NODES, DA), jnp.float32),
      pltpu.VMEM_SHARED((ACC_ROWS, DA), jnp.float32),
      pltpu.SemaphoreType.DMA,
      pltpu.SemaphoreType.DMA,
  ]
  if with_cnt:
    out_type.append(jax.ShapeDtypeStruct((NC, ACC_ROWS), jnp.float32))
    scratch += [
        pltpu.VMEM((CH,), jnp.float32),
        pltpu.VMEM((RPT,), jnp.float32),
        pltpu.VMEM_SHARED((ACC_ROWS,), jnp.float32),
    ]

  n16 = DA // 16

  def body(table, idx_h, *rest):
    zeros16 = jnp.zeros((16,), jnp.float32)
    ones16 = jnp.ones((16,), jnp.float32)
    if with_cnt:
      (out_a, out_c, src_v, dst_v, rows, tab, acc, sem0, sem1,
       ones_v, zc_v, cacc) = rest
    else:
      (out_a, src_v, dst_v, rows, tab, acc, sem0, sem1) = rest
    cid = lax.axis_index("c")
    sid = lax.axis_index("s")
    tid = cid * NS + sid
    base = sid * RPT

    pltpu.async_copy(idx_h.at[tid, 0], src_v, sem0)
    pltpu.async_copy(idx_h.at[tid, 1], dst_v, sem1)
    cs = pl.ds(col_off, DA)
    pltpu.sync_copy(table.at[pl.ds(sid * TST, TST), cs],
                    tab.at[pl.ds(sid * TST, TST)])
    @pl.when(sid == 0)
    def _():
      pltpu.sync_copy(table.at[pl.ds(NS * TST, N_NODES - NS * TST), cs],
                      tab.at[pl.ds(NS * TST, N_NODES - NS * TST)])

    def zrow(r, _):
      for c in range(n16):
        rows[r, pl.ds(c * 16, 16)] = zeros16
      return 0
    lax.fori_loop(0, CH, zrow, 0)
    for k in range(RPT // CH):
      pltpu.sync_copy(rows, acc.at[pl.ds(base + k * CH, CH)])
    if with_cnt:
      def zc(i, _):
        ones_v[pl.ds(i * 16, 16)] = ones16
        return 0
      lax.fori_loop(0, CH // 16, zc, 0)
      def zc2(i, _):
        zc_v[pl.ds(i * 16, 16)] = zeros16
        return 0
      lax.fori_loop(0, RPT // 16, zc2, 0)
      pltpu.sync_copy(zc_v, cacc.at[pl.ds(base, RPT)])
    pltpu.make_async_copy(idx_h.at[tid, 0], src_v, sem0).wait()
    pltpu.make_async_copy(idx_h.at[tid, 1], dst_v, sem1).wait()
    plsc.subcore_barrier()

    def step(j, _):
      pltpu.async_copy(tab.at[src_v.at[j]], rows, sem0).wait()
      if with_cnt:
        pltpu.sync_copy(ones_v, cacc.at[dst_v.at[j]], add=True)
      pltpu.sync_copy(rows, acc.at[dst_v.at[j]], add=True)
      return 0
    lax.fori_loop(0, NCH, step, 0)
    plsc.subcore_barrier()

    pltpu.sync_copy(acc.at[pl.ds(base, RPT)], out_a.at[cid, pl.ds(base, RPT)])
    if with_cnt:
      pltpu.sync_copy(cacc.at[pl.ds(base, RPT)],
                      out_c.at[cid, pl.ds(base, RPT)])

  return pl.kernel(body, out_type=tuple(out_type), mesh=mesh,
                   scratch_types=tuple(scratch),
                   compiler_params=pltpu.CompilerParams(
                       use_tc_tiling_on_sc=False))


_sc_agg_lo = _make_sc_agg(True, 0)
_sc_agg_hi = _make_sc_agg(False, DA)
_sc_agg_p = _make_sc_agg(False, 0)

BR = 1000


def _densea_body(x, w1r, r_ref):
  r_ref[...] = lax.dot_general(x[...], w1r[...], (((1,), (0,)), ((), ())),
                               preferred_element_type=jnp.float32)


def _denseb_body(al, ah, cnt, r, w1l, b1l, w2l, h_ref, p_ref, ic_ref):
  c = jnp.maximum(cnt[0] + cnt[1], 1.0)
  mean = jnp.concatenate([al[0] + al[1], ah[0] + ah[1]], axis=1) / c
  h = lax.dot_general(mean, w1l[...], (((1,), (0,)), ((), ())),
                      preferred_element_type=jnp.float32)
  h = jnp.maximum(h + r[...] + b1l[...], 0.0)
  h_ref[...] = h
  p_ref[...] = lax.dot_general(h, w2l[...], (((1,), (0,)), ((), ())),
                               preferred_element_type=jnp.float32)
  ic_ref[...] = 1.0 / c


def _densec_body(h, w2r, q_ref):
  q_ref[...] = lax.dot_general(h[...], w2r[...], (((1,), (0,)), ((), ())),
                               preferred_element_type=jnp.float32)


def _final_body(g, ic, q, b2l, out_ref):
  out_ref[...] = (g[0] + g[1]) * ic[...] + q[...] + b2l[...]


def _row_blk(d):
  return pl.BlockSpec((BR, d), lambda i: (i, 0))


def _pair_blk(d):
  return pl.BlockSpec((NC, BR, d), lambda i: (0, i, 0))


def _full_blk(r, d):
  return pl.BlockSpec((r, d), lambda i: (0, 0))


_densea_call = pl.pallas_call(
    _densea_body,
    grid=(N_NODES // BR,),
    in_specs=[_row_blk(D_IN), _full_blk(D_IN, D_HID)],
    out_specs=_row_blk(D_HID),
    out_shape=jax.ShapeDtypeStruct((N_NODES, D_HID), jnp.float32),
)

_denseb_call = pl.pallas_call(
    _denseb_body,
    grid=(N_NODES // BR,),
    in_specs=[
        _pair_blk(DA),
        _pair_blk(DA),
        _pair_blk(1),
        _row_blk(D_HID),
        _full_blk(D_IN, D_HID),
        _full_blk(1, D_HID),
        _full_blk(D_HID, D_OUT),
    ],
    out_specs=[
        _row_blk(D_HID),
        _row_blk(D_OUT),
        _row_blk(1),
    ],
    out_shape=[
        jax.ShapeDtypeStruct((N_NODES, D_HID), jnp.float32),
        jax.ShapeDtypeStruct((N_NODES, D_OUT), jnp.float32),
        jax.ShapeDtypeStruct((N_NODES, 1), jnp.float32),
    ],
)

_densec_call = pl.pallas_call(
    _densec_body,
    grid=(N_NODES // BR,),
    in_specs=[_row_blk(D_HID), _full_blk(D_HID, D_OUT)],
    out_specs=_row_blk(D_OUT),
    out_shape=jax.ShapeDtypeStruct((N_NODES, D_OUT), jnp.float32),
)

_final_call = pl.pallas_call(
    _final_body,
    grid=(N_NODES // BR,),
    in_specs=[
        _pair_blk(DA),
        _row_blk(1),
        _row_blk(D_OUT),
        _full_blk(1, D_OUT),
    ],
    out_specs=_row_blk(D_OUT),
    out_shape=jax.ShapeDtypeStruct((N_NODES, D_OUT), jnp.float32),
)


@jax.jit
def _run(x, edge_index, W1l, b1l, W1r, W2l, b2l, W2r):
  src = edge_index[0].astype(jnp.int32)
  dst = edge_index[1].astype(jnp.int32)
  pad = E_PAD - N_EDGES
  src = jnp.concatenate([src, jnp.zeros((pad,), jnp.int32)])
  dst = jnp.concatenate([dst, jnp.full((pad,), N_NODES, jnp.int32)])
  src_r = src.reshape(NT, 1, NCH, CH)
  dst_r = dst.reshape(NT, 1, NCH, CH)
  idx = jnp.concatenate([src_r, dst_r], axis=1)

  al, cnt = _sc_agg_lo(x, idx)
  (ah,) = _sc_agg_hi(x, idx)
  r = _densea_call(x, W1r.T)
  h, p, ic = _denseb_call(al, ah, cnt[:, :, None], r, W1l.T, b1l[None, :],
                          W2l.T)
  (g,) = _sc_agg_p(p, idx)
  q = _densec_call(h, W2r.T)
  return _final_call(g, ic, q, b2l[None, :])


def kernel(x, edge_index, W1l, b1l, W1r, W2l, b2l, W2r):
  return _run(x, edge_index, W1l, b1l, W1r, W2l, b2l, W2r)


# speedup vs baseline: 1.3795x; 1.2458x over previous
import functools

import jax
import jax.numpy as jnp
from jax import lax
from jax.experimental import pallas as pl
from jax.experimental.pallas import tpu as pltpu
from jax.experimental.pallas import tpu_sc as plsc

N_NODES = 10000
N_EDGES = 320000
D_IN = 128
D_HID = 256
D_OUT = 64
DA = 64

NC = 2
NS = 16
NT = NC * NS
CH = 128
NCH = 2 * -(-N_EDGES // (NT * CH * 2))
E_PAD = NT * NCH * CH
RPT = 640
ACC_ROWS = NS * RPT
TST = 624


def _make_sc_agg(with_cnt, col_off=0, in_cols=DA):
  mesh = plsc.VectorSubcoreMesh(core_axis_name="c", subcore_axis_name="s")
  out_type = [jax.ShapeDtypeStruct((NC, ACC_ROWS, DA), jnp.float32)]
  scratch = [
      pltpu.VMEM((NCH, CH), jnp.int32),
      pltpu.VMEM((NCH, CH), jnp.int32),
      pltpu.VMEM((CH, DA), jnp.float32),
      pltpu.VMEM((CH, DA), jnp.float32),
      pltpu.VMEM_SHARED((N_NODES, DA), jnp.float32),
      pltpu.VMEM_SHARED((ACC_ROWS, DA), jnp.float32),
      pltpu.SemaphoreType.DMA,
      pltpu.SemaphoreType.DMA,
  ]
  if with_cnt:
    out_type.append(jax.ShapeDtypeStruct((NC, ACC_ROWS), jnp.float32))
    scratch += [
        pltpu.VMEM((CH,), jnp.float32),
        pltpu.VMEM((RPT,), jnp.float32),
        pltpu.VMEM_SHARED((ACC_ROWS,), jnp.float32),
    ]

  n16 = DA // 16

  def body(table, idx_h, *rest):
    zeros16 = jnp.zeros((16,), jnp.float32)
    ones16 = jnp.ones((16,), jnp.float32)
    if with_cnt:
      (out_a, out_c, src_v, dst_v, rows, rows1, tab, acc, sem0, sem1,
       ones_v, zc_v, cacc) = rest
    else:
      (out_a, src_v, dst_v, rows, rows1, tab, acc, sem0, sem1) = rest
    cid = lax.axis_index("c")
    sid = lax.axis_index("s")
    tid = cid * NS + sid
    base = sid * RPT

    pltpu.async_copy(idx_h.at[tid, 0], src_v, sem0)
    pltpu.async_copy(idx_h.at[tid, 1], dst_v, sem1)
    cs = pl.ds(col_off, DA)
    pltpu.sync_copy(table.at[pl.ds(sid * TST, TST), cs],
                    tab.at[pl.ds(sid * TST, TST)])
    @pl.when(sid == 0)
    def _():
      pltpu.sync_copy(table.at[pl.ds(NS * TST, N_NODES - NS * TST), cs],
                      tab.at[pl.ds(NS * TST, N_NODES - NS * TST)])

    def zrow(r, _):
      for c in range(n16):
        rows[r, pl.ds(c * 16, 16)] = zeros16
      return 0
    lax.fori_loop(0, CH, zrow, 0)
    for k in range(RPT // CH):
      pltpu.sync_copy(rows, acc.at[pl.ds(base + k * CH, CH)])
    if with_cnt:
      def zc(i, _):
        ones_v[pl.ds(i * 16, 16)] = ones16
        return 0
      lax.fori_loop(0, CH // 16, zc, 0)
      def zc2(i, _):
        zc_v[pl.ds(i * 16, 16)] = zeros16
        return 0
      lax.fori_loop(0, RPT // 16, zc2, 0)
      pltpu.sync_copy(zc_v, cacc.at[pl.ds(base, RPT)])
    pltpu.make_async_copy(idx_h.at[tid, 0], src_v, sem0).wait()
    pltpu.make_async_copy(idx_h.at[tid, 1], dst_v, sem1).wait()
    plsc.subcore_barrier()

    pltpu.async_copy(tab.at[src_v.at[0]], rows, sem0)

    def step(i, _):
      j = 2 * i
      pltpu.async_copy(tab.at[src_v.at[j + 1]], rows1, sem1)
      pltpu.make_async_copy(tab.at[src_v.at[j]], rows, sem0).wait()
      if with_cnt:
        pltpu.sync_copy(ones_v, cacc.at[dst_v.at[j]], add=True)
      pltpu.sync_copy(rows, acc.at[dst_v.at[j]], add=True)
      @pl.when(i < NCH // 2 - 1)
      def _():
        pltpu.async_copy(tab.at[src_v.at[j + 2]], rows, sem0)
      pltpu.make_async_copy(tab.at[src_v.at[j + 1]], rows1, sem1).wait()
      if with_cnt:
        pltpu.sync_copy(ones_v, cacc.at[dst_v.at[j + 1]], add=True)
      pltpu.sync_copy(rows1, acc.at[dst_v.at[j + 1]], add=True)
      return 0
    lax.fori_loop(0, NCH // 2, step, 0)
    plsc.subcore_barrier()

    pltpu.sync_copy(acc.at[pl.ds(base, RPT)], out_a.at[cid, pl.ds(base, RPT)])
    if with_cnt:
      pltpu.sync_copy(cacc.at[pl.ds(base, RPT)],
                      out_c.at[cid, pl.ds(base, RPT)])

  return pl.kernel(body, out_type=tuple(out_type), mesh=mesh,
                   scratch_types=tuple(scratch),
                   compiler_params=pltpu.CompilerParams(
                       use_tc_tiling_on_sc=False))


_sc_agg_lo = _make_sc_agg(True, 0)
_sc_agg_hi = _make_sc_agg(False, DA)
_sc_agg_p = _make_sc_agg(False, 0)

BR = 1000


def _densea_body(x, w1r, r_ref):
  r_ref[...] = lax.dot_general(x[...], w1r[...], (((1,), (0,)), ((), ())),
                               preferred_element_type=jnp.float32)


def _denseb_body(al, ah, cnt, r, w1l, b1l, w2l, h_ref, p_ref, ic_ref):
  c = jnp.maximum(cnt[0] + cnt[1], 1.0)
  mean = jnp.concatenate([al[0] + al[1], ah[0] + ah[1]], axis=1) / c
  h = lax.dot_general(mean, w1l[...], (((1,), (0,)), ((), ())),
                      preferred_element_type=jnp.float32)
  h = jnp.maximum(h + r[...] + b1l[...], 0.0)
  h_ref[...] = h
  p_ref[...] = lax.dot_general(h, w2l[...], (((1,), (0,)), ((), ())),
                               preferred_element_type=jnp.float32)
  ic_ref[...] = 1.0 / c


def _densec_body(h, w2r, q_ref):
  q_ref[...] = lax.dot_general(h[...], w2r[...], (((1,), (0,)), ((), ())),
                               preferred_element_type=jnp.float32)


def _final_body(g, ic, q, b2l, out_ref):
  out_ref[...] = (g[0] + g[1]) * ic[...] + q[...] + b2l[...]


def _row_blk(d):
  return pl.BlockSpec((BR, d), lambda i: (i, 0))


def _pair_blk(d):
  return pl.BlockSpec((NC, BR, d), lambda i: (0, i, 0))


def _full_blk(r, d):
  return pl.BlockSpec((r, d), lambda i: (0, 0))


_densea_call = pl.pallas_call(
    _densea_body,
    grid=(N_NODES // BR,),
    in_specs=[_row_blk(D_IN), _full_blk(D_IN, D_HID)],
    out_specs=_row_blk(D_HID),
    out_shape=jax.ShapeDtypeStruct((N_NODES, D_HID), jnp.float32),
)

_denseb_call = pl.pallas_call(
    _denseb_body,
    grid=(N_NODES // BR,),
    in_specs=[
        _pair_blk(DA),
        _pair_blk(DA),
        _pair_blk(1),
        _row_blk(D_HID),
        _full_blk(D_IN, D_HID),
        _full_blk(1, D_HID),
        _full_blk(D_HID, D_OUT),
    ],
    out_specs=[
        _row_blk(D_HID),
        _row_blk(D_OUT),
        _row_blk(1),
    ],
    out_shape=[
        jax.ShapeDtypeStruct((N_NODES, D_HID), jnp.float32),
        jax.ShapeDtypeStruct((N_NODES, D_OUT), jnp.float32),
        jax.ShapeDtypeStruct((N_NODES, 1), jnp.float32),
    ],
)

_densec_call = pl.pallas_call(
    _densec_body,
    grid=(N_NODES // BR,),
    in_specs=[_row_blk(D_HID), _full_blk(D_HID, D_OUT)],
    out_specs=_row_blk(D_OUT),
    out_shape=jax.ShapeDtypeStruct((N_NODES, D_OUT), jnp.float32),
)

_final_call = pl.pallas_call(
    _final_body,
    grid=(N_NODES // BR,),
    in_specs=[
        _pair_blk(DA),
        _row_blk(1),
        _row_blk(D_OUT),
        _full_blk(1, D_OUT),
    ],
    out_specs=_row_blk(D_OUT),
    out_shape=jax.ShapeDtypeStruct((N_NODES, D_OUT), jnp.float32),
)


@jax.jit
def _run(x, edge_index, W1l, b1l, W1r, W2l, b2l, W2r):
  src = edge_index[0].astype(jnp.int32)
  dst = edge_index[1].astype(jnp.int32)
  pad = E_PAD - N_EDGES
  src = jnp.concatenate([src, jnp.zeros((pad,), jnp.int32)])
  dst = jnp.concatenate([dst, jnp.full((pad,), N_NODES, jnp.int32)])
  src_r = src.reshape(NT, 1, NCH, CH)
  dst_r = dst.reshape(NT, 1, NCH, CH)
  idx = jnp.concatenate([src_r, dst_r], axis=1)

  al, cnt = _sc_agg_lo(x, idx)
  (ah,) = _sc_agg_hi(x, idx)
  r = _densea_call(x, W1r.T)
  h, p, ic = _denseb_call(al, ah, cnt[:, :, None], r, W1l.T, b1l[None, :],
                          W2l.T)
  (g,) = _sc_agg_p(p, idx)
  q = _densec_call(h, W2r.T)
  return _final_call(g, ic, q, b2l[None, :])


def kernel(x, edge_index, W1l, b1l, W1r, W2l, b2l, W2r):
  return _run(x, edge_index, W1l, b1l, W1r, W2l, b2l, W2r)
